# baseline (speedup 1.0000x reference)
import functools

import jax
import jax.numpy as jnp
from jax import lax
from jax.experimental import pallas as pl
from jax.experimental.pallas import tpu as pltpu
from jax.experimental.pallas import tpu_sc as plsc


def _gather_call(B, D, NC, NS, b_per_w, ch):
    mesh = plsc.VectorSubcoreMesh(core_axis_name="c", subcore_axis_name="s")
    n_chunks = b_per_w // ch
    n_groups = b_per_w // 16
    lanes = 4 * D

    @functools.partial(
        pl.kernel,
        mesh=mesh,
        out_type=jax.ShapeDtypeStruct((B, D), jnp.float32),
        scratch_types=[
            pltpu.VMEM((b_per_w,), jnp.int32),
            pltpu.VMEM((n_chunks, ch), jnp.int32),
            pltpu.VMEM((b_per_w, lanes), jnp.float32),
            pltpu.VMEM((b_per_w, D), jnp.float32),
            pltpu.SemaphoreType.DMA,
        ],
        compiler_params=pltpu.CompilerParams(use_tc_tiling_on_sc=False),
    )
    def gather_kernel(idx_hbm, t4_hbm, out_hbm, idx_v, rec_v, grp_v, rows_v, sem):
        wid = lax.axis_index("s") * NC + lax.axis_index("c")
        base = wid * b_per_w
        pltpu.sync_copy(idx_hbm.at[wid], idx_v)

        groups_per_chunk = ch // 16

        def rec_ids(g, carry):
            iv = idx_v[pl.ds(g * 16, 16)]
            j = g // groups_per_chunk
            o = (g % groups_per_chunk) * 16
            rec_v[j, pl.ds(o, 16)] = lax.shift_right_logical(iv, 2)
            return carry

        lax.fori_loop(0, n_groups, rec_ids, 0)

        copies = []
        for j in range(n_chunks):
            copies.append(
                pltpu.make_async_copy(
                    t4_hbm.at[rec_v.at[j]],
                    grp_v.at[pl.ds(j * ch, ch), :],
                    sem,
                )
            )
            copies[-1].start()
        for c in copies:
            c.wait()

        def extract(g, carry):
            iv = idx_v[pl.ds(g * 16, 16)]
            qv = lax.shift_left(lax.bitwise_and(iv, 3), 5)
            for j in range(16):
                b = g * 16 + j
                off = qv[j]
                rows_v[b, pl.ds(0, 16)] = grp_v[b, pl.ds(off, 16)]
                rows_v[b, pl.ds(16, 16)] = grp_v[b, pl.ds(off + 16, 16)]
            return carry

        lax.fori_loop(0, n_groups, extract, 0)
        pltpu.sync_copy(rows_v, out_hbm.at[pl.ds(base, b_per_w), :])

    return gather_kernel


def kernel(indices, table):
    B = indices.shape[0]
    V, D = table.shape
    info = plsc.get_sparse_core_info()
    NC, NS = info.num_cores, info.num_subcores
    NW = NC * NS
    b_per_w = B // NW
    ch = 128

    idx2 = indices.astype(jnp.int32).reshape(NW, b_per_w)
    t4 = table.reshape(V // 4, 4 * D)
    call = _gather_call(B, D, NC, NS, b_per_w, ch)
    return call(idx2, t4)


# speedup vs baseline: 1.0098x; 1.0098x over previous
import functools

import jax
import jax.numpy as jnp
from jax import lax
from jax.experimental import pallas as pl
from jax.experimental.pallas import tpu as pltpu
from jax.experimental.pallas import tpu_sc as plsc


def _gather_call(B, D, NC, NS, b_per_w, n_chunks, ch):
    mesh = plsc.VectorSubcoreMesh(core_axis_name="c", subcore_axis_name="s")

    @functools.partial(
        pl.kernel,
        mesh=mesh,
        out_type=jax.ShapeDtypeStruct((B, D), jnp.float32),
        scratch_types=[
            pltpu.VMEM((n_chunks, ch), jnp.int32),
            pltpu.VMEM((b_per_w, D), jnp.float32),
            pltpu.SemaphoreType.DMA,
        ],
        compiler_params=pltpu.CompilerParams(use_tc_tiling_on_sc=False),
    )
    def gather_kernel(idx_hbm, table_hbm, out_hbm, idx_v, rows_v, sem):
        wid = lax.axis_index("s") * NC + lax.axis_index("c")
        base = wid * b_per_w
        pltpu.sync_copy(idx_hbm.at[wid], idx_v)
        copies = []
        for j in range(n_chunks):
            copies.append(
                pltpu.make_async_copy(
                    table_hbm.at[idx_v.at[j]],
                    rows_v.at[pl.ds(j * ch, ch), :],
                    sem,
                )
            )
            copies[-1].start()
        for c in copies:
            c.wait()
        pltpu.sync_copy(rows_v, out_hbm.at[pl.ds(base, b_per_w)])

    return gather_kernel


def kernel(indices, table):
    B = indices.shape[0]
    V, D = table.shape
    info = plsc.get_sparse_core_info()
    NC, NS = info.num_cores, info.num_subcores
    NW = NC * NS
    b_per_w = B // NW
    ch = 128
    n_chunks = b_per_w // ch

    idx3 = indices.astype(jnp.int32).reshape(NW, n_chunks, ch)
    call = _gather_call(B, D, NC, NS, b_per_w, n_chunks, ch)
    return call(idx3, table)
